# baseline (speedup 1.0000x reference)
import jax
import jax.numpy as jnp
from jax.experimental import pallas as pl
from jax.experimental.pallas import tpu as pltpu

_N = 128
_C = 32768
_RB = 32
_NK = _N // _RB
_NS = 5


def _ste_body(x_hbm, o_hbm, in_buf, out_buf, in_sem, out_sem):
    def get_copy(k, slot):
        return pltpu.make_async_copy(
            x_hbm.at[pl.ds(k * _RB, _RB), :], in_buf.at[slot], in_sem.at[slot])

    def put_copy(k, slot):
        return pltpu.make_async_copy(
            out_buf.at[slot], o_hbm.at[pl.ds(k * _RB, _RB), :], out_sem.at[slot])

    for k in range(_NS - 1):
        get_copy(k, k % _NS).start()
    for k in range(_NK):
        slot = k % _NS
        if k + _NS - 1 < _NK:
            get_copy(k + _NS - 1, (k + _NS - 1) % _NS).start()
        get_copy(k, slot).wait()
        xb = in_buf[slot]
        idx = jnp.argmax(xb, axis=1)
        ii = jax.lax.broadcasted_iota(jnp.int32, (_RB, _C), 1)
        if k >= _NS:
            put_copy(k - _NS, slot).wait()
        out_buf[slot] = (ii == idx[:, None]).astype(jnp.float32)
        put_copy(k, slot).start()
    for k in range(max(_NK - _NS, 0), _NK):
        put_copy(k, k % _NS).wait()


@jax.jit
def kernel(x):
    return pl.pallas_call(
        _ste_body,
        in_specs=[pl.BlockSpec(memory_space=pl.MemorySpace.ANY)],
        out_specs=pl.BlockSpec(memory_space=pl.MemorySpace.ANY),
        out_shape=jax.ShapeDtypeStruct((_N, _C), jnp.float32),
        scratch_shapes=[
            pltpu.VMEM((_NS, _RB, _C), jnp.float32),
            pltpu.VMEM((_NS, _RB, _C), jnp.float32),
            pltpu.SemaphoreType.DMA((_NS,)),
            pltpu.SemaphoreType.DMA((_NS,)),
        ],
    )(x)
